# baseline (speedup 1.0000x reference)
import functools

import jax
import jax.numpy as jnp
from jax import lax
from jax.experimental import pallas as pl
from jax.experimental.pallas import tpu as pltpu
from jax.experimental.pallas import tpu_sc as plsc

_HIDDEN = 2048
_BATCH = 16384
_NC = 2
_NS = 16
_NW = _NC * _NS
_SC_ROWS = 4096
_TC_ROWS = _BATCH - _SC_ROWS
_ROWS_PER_W = _SC_ROWS // _NW
_REP = 32
_N_DMA = _ROWS_PER_W // _REP
_TC_BLOCK = 1024

_mesh = plsc.VectorSubcoreMesh(core_axis_name="c", subcore_axis_name="s")


@functools.partial(
    pl.kernel,
    out_type=jax.ShapeDtypeStruct((_SC_ROWS, _HIDDEN), jnp.float32),
    mesh=_mesh,
    scratch_types=[
        pltpu.VMEM((_REP, _HIDDEN), jnp.float32),
        pltpu.SemaphoreType.DMA,
    ],
)
def _sc_broadcast(w_hbm, out_hbm, buf, sem):
    wid = lax.axis_index("c") * _NS + lax.axis_index("s")
    base = wid * _ROWS_PER_W
    fills = [
        pltpu.async_copy(w_hbm, buf.at[pl.ds(i, 1)], sem) for i in range(_REP)
    ]
    for f in fills:
        f.wait()
    copies = [
        pltpu.async_copy(buf, out_hbm.at[pl.ds(base + i * _REP, _REP)], sem)
        for i in range(_N_DMA)
    ]
    for c in copies:
        c.wait()


def _tc_body(w_ref, out_ref):
    out_ref[...] = jnp.broadcast_to(w_ref[...], out_ref.shape)


_tc_broadcast = pl.pallas_call(
    _tc_body,
    grid=(_TC_ROWS // _TC_BLOCK,),
    in_specs=[pl.BlockSpec((1, _HIDDEN), lambda i: (0, 0))],
    out_specs=pl.BlockSpec((_TC_BLOCK, _HIDDEN), lambda i: (i, 0)),
    out_shape=jax.ShapeDtypeStruct((_TC_ROWS, _HIDDEN), jnp.float32),
)


def kernel(x, embedding_weight):
    del x
    sc_half = _sc_broadcast(embedding_weight)
    tc_half = _tc_broadcast(embedding_weight)
    return jnp.concatenate([tc_half, sc_half], axis=0)


# speedup vs baseline: 2.1143x; 2.1143x over previous
import functools

import jax
import jax.numpy as jnp
from jax import lax
from jax.experimental import pallas as pl
from jax.experimental.pallas import tpu as pltpu
from jax.experimental.pallas import tpu_sc as plsc

_HIDDEN = 2048
_BATCH = 16384
_NC = 2
_NS = 16
_NW = _NC * _NS
_ROWS_PER_W = _BATCH // _NW
_REP = 16
_N_DMA = _ROWS_PER_W // _REP

_mesh = plsc.VectorSubcoreMesh(core_axis_name="c", subcore_axis_name="s")


@functools.partial(
    pl.kernel,
    out_type=jax.ShapeDtypeStruct((_BATCH, _HIDDEN), jnp.float32),
    mesh=_mesh,
    scratch_types=[
        pltpu.VMEM((_REP, _HIDDEN), jnp.float32),
        pltpu.SemaphoreType.DMA,
    ],
)
def _broadcast_row(w_hbm, out_hbm, buf, sem):
    wid = lax.axis_index("c") * _NS + lax.axis_index("s")
    base = wid * _ROWS_PER_W
    fills = [
        pltpu.async_copy(w_hbm, buf.at[pl.ds(i, 1)], sem) for i in range(_REP)
    ]
    for f in fills:
        f.wait()
    copies = [
        pltpu.async_copy(buf, out_hbm.at[pl.ds(base + i * _REP, _REP)], sem)
        for i in range(_N_DMA)
    ]
    for c in copies:
        c.wait()


def kernel(x, embedding_weight):
    del x
    return _broadcast_row(embedding_weight)


# speedup vs baseline: 2.2511x; 1.0647x over previous
import functools

import jax
import jax.numpy as jnp
from jax import lax
from jax.experimental import pallas as pl
from jax.experimental.pallas import tpu as pltpu
from jax.experimental.pallas import tpu_sc as plsc

_HIDDEN = 2048
_BATCH = 16384
_NC = 2
_NS = 16
_NW = _NC * _NS
_ROWS_PER_W = _BATCH // _NW
_REP = 8
_N_DMA = _ROWS_PER_W // _REP

_mesh = plsc.VectorSubcoreMesh(core_axis_name="c", subcore_axis_name="s")


@functools.partial(
    pl.kernel,
    out_type=jax.ShapeDtypeStruct((_BATCH, _HIDDEN), jnp.float32),
    mesh=_mesh,
    scratch_types=[
        pltpu.VMEM((_REP, _HIDDEN), jnp.float32),
        pltpu.SemaphoreType.DMA,
    ],
)
def _broadcast_row(w_hbm, out_hbm, buf, sem):
    wid = lax.axis_index("c") * _NS + lax.axis_index("s")
    base = wid * _ROWS_PER_W
    fills = [
        pltpu.async_copy(w_hbm, buf.at[pl.ds(i, 1)], sem) for i in range(_REP)
    ]
    for f in fills:
        f.wait()
    copies = [
        pltpu.async_copy(buf, out_hbm.at[pl.ds(base + i * _REP, _REP)], sem)
        for i in range(_N_DMA)
    ]
    for c in copies:
        c.wait()


def kernel(x, embedding_weight):
    del x
    return _broadcast_row(embedding_weight)


# speedup vs baseline: 2.4555x; 1.0908x over previous
import functools

import jax
import jax.numpy as jnp
from jax import lax
from jax.experimental import pallas as pl
from jax.experimental.pallas import tpu as pltpu
from jax.experimental.pallas import tpu_sc as plsc

_HIDDEN = 2048
_BATCH = 16384
_NC = 2
_NS = 16
_NW = _NC * _NS
_ROWS_PER_W = _BATCH // _NW
_REP = 4
_N_DMA = _ROWS_PER_W // _REP

_mesh = plsc.VectorSubcoreMesh(core_axis_name="c", subcore_axis_name="s")


@functools.partial(
    pl.kernel,
    out_type=jax.ShapeDtypeStruct((_BATCH, _HIDDEN), jnp.float32),
    mesh=_mesh,
    scratch_types=[
        pltpu.VMEM((_REP, _HIDDEN), jnp.float32),
        pltpu.SemaphoreType.DMA,
    ],
)
def _broadcast_row(w_hbm, out_hbm, buf, sem):
    wid = lax.axis_index("c") * _NS + lax.axis_index("s")
    base = wid * _ROWS_PER_W
    fills = [
        pltpu.async_copy(w_hbm, buf.at[pl.ds(i, 1)], sem) for i in range(_REP)
    ]
    for f in fills:
        f.wait()
    copies = [
        pltpu.async_copy(buf, out_hbm.at[pl.ds(base + i * _REP, _REP)], sem)
        for i in range(_N_DMA)
    ]
    for c in copies:
        c.wait()


def kernel(x, embedding_weight):
    del x
    return _broadcast_row(embedding_weight)


# speedup vs baseline: 2.7466x; 1.1185x over previous
import functools

import jax
import jax.numpy as jnp
from jax import lax
from jax.experimental import pallas as pl
from jax.experimental.pallas import tpu as pltpu
from jax.experimental.pallas import tpu_sc as plsc

_HIDDEN = 2048
_BATCH = 16384
_NC = 2
_NS = 16
_NW = _NC * _NS
_ROWS_PER_W = _BATCH // _NW
_REP = 2
_N_DMA = _ROWS_PER_W // _REP

_mesh = plsc.VectorSubcoreMesh(core_axis_name="c", subcore_axis_name="s")


@functools.partial(
    pl.kernel,
    out_type=jax.ShapeDtypeStruct((_BATCH, _HIDDEN), jnp.float32),
    mesh=_mesh,
    scratch_types=[
        pltpu.VMEM((_REP, _HIDDEN), jnp.float32),
        pltpu.SemaphoreType.DMA,
    ],
)
def _broadcast_row(w_hbm, out_hbm, buf, sem):
    wid = lax.axis_index("c") * _NS + lax.axis_index("s")
    base = wid * _ROWS_PER_W
    fills = [
        pltpu.async_copy(w_hbm, buf.at[pl.ds(i, 1)], sem) for i in range(_REP)
    ]
    for f in fills:
        f.wait()
    copies = [
        pltpu.async_copy(buf, out_hbm.at[pl.ds(base + i * _REP, _REP)], sem)
        for i in range(_N_DMA)
    ]
    for c in copies:
        c.wait()


def kernel(x, embedding_weight):
    del x
    return _broadcast_row(embedding_weight)


# speedup vs baseline: 2.7828x; 1.0132x over previous
import functools

import jax
import jax.numpy as jnp
from jax import lax
from jax.experimental import pallas as pl
from jax.experimental.pallas import tpu as pltpu
from jax.experimental.pallas import tpu_sc as plsc

_HIDDEN = 2048
_BATCH = 16384
_NC = 2
_NS = 16
_NW = _NC * _NS
_ROWS_PER_W = _BATCH // _NW
_REP = 1
_N_DMA = _ROWS_PER_W // _REP

_mesh = plsc.VectorSubcoreMesh(core_axis_name="c", subcore_axis_name="s")


@functools.partial(
    pl.kernel,
    out_type=jax.ShapeDtypeStruct((_BATCH, _HIDDEN), jnp.float32),
    mesh=_mesh,
    scratch_types=[
        pltpu.VMEM((_REP, _HIDDEN), jnp.float32),
        pltpu.SemaphoreType.DMA,
    ],
)
def _broadcast_row(w_hbm, out_hbm, buf, sem):
    wid = lax.axis_index("c") * _NS + lax.axis_index("s")
    base = wid * _ROWS_PER_W
    fills = [
        pltpu.async_copy(w_hbm, buf.at[pl.ds(i, 1)], sem) for i in range(_REP)
    ]
    for f in fills:
        f.wait()
    copies = [
        pltpu.async_copy(buf, out_hbm.at[pl.ds(base + i * _REP, _REP)], sem)
        for i in range(_N_DMA)
    ]
    for c in copies:
        c.wait()


def kernel(x, embedding_weight):
    del x
    return _broadcast_row(embedding_weight)
